# baseline (speedup 1.0000x reference)
import jax
import jax.numpy as jnp
from jax.experimental import pallas as pl

_NUM_CLUSTERS = 8
_EPS = 1e-05
_BLOCK_ROWS = 512

_SQRT_2_OVER_PI = 0.7978845608028654
_LOG2E = 1.4426950408889634
_LN2 = 0.6931471805599453
_BIG = 3.0e38


def _body(x_ref, c_ref, c2_ref, ones_ref, o_ref):
    xb = x_ref[...]
    cb = c_ref[...]
    ones = ones_ref[...]
    d = xb.shape[1]

    cdims = (((1,), (1,)), ((), ()))
    s1 = jax.lax.dot_general(xb, ones, cdims, preferred_element_type=jnp.float32)
    xsq = xb * xb
    s2 = jax.lax.dot_general(xsq, ones, cdims, preferred_element_type=jnp.float32)

    dots = jax.lax.dot_general(xb, cb, cdims, preferred_element_type=jnp.float32)
    dist = s2 - 2.0 * dots + c2_ref[...]

    mind = jnp.min(dist, axis=1, keepdims=True)
    lane = jax.lax.broadcasted_iota(jnp.int32, dist.shape, 1)
    lab = jnp.min(
        jnp.where(dist == mind, lane, _NUM_CLUSTERS), axis=1, keepdims=True
    )

    mean = s1 * (1.0 / d)
    var = (s2 - s1 * mean) * (1.0 / (d - 1))
    rstd = jax.lax.rsqrt(var + _EPS)
    xn = (xb - mean) * rstd

    ha = jnp.where(
        lab == 1, _SQRT_2_OVER_PI,
        jnp.where(lab == 2, 1.0, jnp.where((lab == 3) | (lab == 4), 0.5, 0.0)),
    )
    hb = jnp.where(lab == 1, _SQRT_2_OVER_PI * 0.044715, 0.0)
    fp = jnp.where((lab == 1) | (lab == 3), 0.5, 0.0)
    fq = jnp.where(lab == 2, 1.0, jnp.where(lab == 4, 0.5, 0.0))
    ft = jnp.where(lab == 2, -1.0, 0.0)
    upper = jnp.where(lab == 5, 6.0, _BIG)
    ek = jnp.where(lab >= 6, _LOG2E, 0.0)
    is_sig = (lab >= 1) & (lab <= 4)
    is_elu = lab == 6
    is_sp = lab == 7

    xnsq = xn * xn
    th = jnp.tanh(xn * (ha + hb * xnsq))
    sig_out = (fp * xn + fq) * (th + 1.0) + ft

    relu = jnp.maximum(xn, 0.0)
    pwl = jnp.minimum(relu, upper)

    em = jnp.exp2(ek * xn)
    elu_out = jnp.where(xn > 0.0, xn, em - 1.0)
    sp_out = jnp.log(1.0 + em)

    out = jnp.where(is_sig, sig_out, pwl)
    out = jnp.where(is_elu, elu_out, out)
    out = jnp.where(is_sp, sp_out, out)
    o_ref[...] = out


@jax.jit
def kernel(x, centroids):
    n, d = x.shape
    c2 = jnp.sum(centroids * centroids, axis=-1)[None, :]
    ones = jnp.ones((1, d), dtype=x.dtype)
    grid = (n // _BLOCK_ROWS,)
    return pl.pallas_call(
        _body,
        grid=grid,
        in_specs=[
            pl.BlockSpec((_BLOCK_ROWS, d), lambda i: (i, 0)),
            pl.BlockSpec((_NUM_CLUSTERS, d), lambda i: (0, 0)),
            pl.BlockSpec((1, _NUM_CLUSTERS), lambda i: (0, 0)),
            pl.BlockSpec((1, d), lambda i: (0, 0)),
        ],
        out_specs=pl.BlockSpec((_BLOCK_ROWS, d), lambda i: (i, 0)),
        out_shape=jax.ShapeDtypeStruct((n, d), x.dtype),
    )(x, centroids, c2, ones)


# speedup vs baseline: 1.0630x; 1.0630x over previous
import jax
import jax.numpy as jnp
from jax.experimental import pallas as pl

_NUM_CLUSTERS = 8
_EPS = 1e-05
_BLOCK_ROWS = 512

_SQRT_2_OVER_PI = 0.7978845608028654
_LOG2E = 1.4426950408889634
_LN2 = 0.6931471805599453
_BIG = 3.0e38


def _body(x_ref, c_ref, c2_ref, o_ref):
    xb = x_ref[...]
    cb = c_ref[...]
    d = xb.shape[1]

    cdims = (((1,), (1,)), ((), ()))
    s1 = jnp.sum(xb, axis=1, keepdims=True)
    s2 = jnp.sum(xb * xb, axis=1, keepdims=True)

    dots = jax.lax.dot_general(xb, cb, cdims, preferred_element_type=jnp.float32)
    dist = s2 - 2.0 * dots + c2_ref[...]

    mind = jnp.min(dist, axis=1, keepdims=True)
    lane = jax.lax.broadcasted_iota(jnp.int32, dist.shape, 1)
    lab = jnp.min(
        jnp.where(dist == mind, lane, _NUM_CLUSTERS), axis=1, keepdims=True
    )

    mean = s1 * (1.0 / d)
    var = (s2 - s1 * mean) * (1.0 / (d - 1))
    rstd = jax.lax.rsqrt(var + _EPS)
    xn = (xb - mean) * rstd

    ha = jnp.where(
        lab == 1, _SQRT_2_OVER_PI,
        jnp.where(lab == 2, 1.0, jnp.where((lab == 3) | (lab == 4), 0.5, 0.0)),
    )
    hb = jnp.where(lab == 1, _SQRT_2_OVER_PI * 0.044715, 0.0)
    fp = jnp.where((lab == 1) | (lab == 3), 0.5, 0.0)
    fq = jnp.where(lab == 2, 1.0, jnp.where(lab == 4, 0.5, 0.0))
    ft = jnp.where(lab == 2, -1.0, 0.0)
    upper = jnp.where(lab == 5, 6.0, _BIG)
    ek = jnp.where(lab >= 6, _LOG2E, 0.0)
    is_sig = (lab >= 1) & (lab <= 4)
    is_elu = lab == 6
    is_sp = lab == 7

    xnsq = xn * xn
    th = jnp.tanh(xn * (ha + hb * xnsq))
    sig_out = (fp * xn + fq) * (th + 1.0) + ft

    relu = jnp.maximum(xn, 0.0)
    pwl = jnp.minimum(relu, upper)

    em = jnp.exp2(ek * xn)
    elu_out = jnp.where(xn > 0.0, xn, em - 1.0)
    sp_out = jnp.log(1.0 + em)

    out = jnp.where(is_sig, sig_out, pwl)
    out = jnp.where(is_elu, elu_out, out)
    out = jnp.where(is_sp, sp_out, out)
    o_ref[...] = out


@jax.jit
def kernel(x, centroids):
    n, d = x.shape
    c2 = jnp.sum(centroids * centroids, axis=-1)[None, :]
    grid = (n // _BLOCK_ROWS,)
    return pl.pallas_call(
        _body,
        grid=grid,
        in_specs=[
            pl.BlockSpec((_BLOCK_ROWS, d), lambda i: (i, 0)),
            pl.BlockSpec((_NUM_CLUSTERS, d), lambda i: (0, 0)),
            pl.BlockSpec((1, _NUM_CLUSTERS), lambda i: (0, 0)),
        ],
        out_specs=pl.BlockSpec((_BLOCK_ROWS, d), lambda i: (i, 0)),
        out_shape=jax.ShapeDtypeStruct((n, d), x.dtype),
    )(x, centroids, c2)
